# baseline (speedup 1.0000x reference)
import functools

import jax
import jax.numpy as jnp
from jax import lax
from jax.experimental import pallas as pl
from jax.experimental.pallas import tpu as pltpu
from jax.experimental.pallas import tpu_sc as plsc

N_TOK = 8192
N_CODE = 8192
DIM = 32

BT = 256
WC = 1024
GT = N_TOK // BT
GC = N_CODE // WC


def _argmin_body(x_ref, a_ref, b_ref, e_ref, idx_ref, loss_ref, mn_ref, mi_ref):
    c = pl.program_id(1)

    @pl.when(c == 0)
    def _init():
        mn_ref[...] = jnp.full((BT, 1), jnp.inf, jnp.float32)
        mi_ref[...] = jnp.zeros((BT, 1), jnp.int32)

    x = x_ref[...]
    a = a_ref[...]
    b = b_ref[...]
    e = e_ref[...]
    cmat = lax.dot_general(x, e, (((1,), (1,)), ((), ())),
                           preferred_element_type=jnp.float32)
    d = (a + b) - 2.0 * cmat
    m = jnp.min(d, axis=1, keepdims=True)
    cols = lax.broadcasted_iota(jnp.int32, (BT, WC), 1) + c * WC
    im = jnp.min(jnp.where(d == m, cols, jnp.int32(2**30)),
                 axis=1, keepdims=True)
    mn = mn_ref[...]
    mi = mi_ref[...]
    upd = m < mn
    mn2 = jnp.where(upd, m, mn)
    mi2 = jnp.where(upd, im, mi)
    mn_ref[...] = mn2
    mi_ref[...] = mi2
    idx_ref[...] = mi2

    @pl.when(jnp.logical_and(pl.program_id(0) == 0, c == 0))
    def _zero():
        loss_ref[...] = jnp.zeros((1, 1), jnp.float32)

    @pl.when(c == GC - 1)
    def _acc():
        loss_ref[...] = loss_ref[...] + jnp.sum(mn2, keepdims=True)


@jax.jit
def _argmin_call(xf, a, b2, emb):
    return pl.pallas_call(
        _argmin_body,
        grid=(GT, GC),
        in_specs=[
            pl.BlockSpec((BT, DIM), lambda i, c: (i, 0)),
            pl.BlockSpec((BT, 1), lambda i, c: (i, 0)),
            pl.BlockSpec((1, WC), lambda i, c: (0, c)),
            pl.BlockSpec((WC, DIM), lambda i, c: (c, 0)),
        ],
        out_specs=[
            pl.BlockSpec((BT, 1), lambda i, c: (i, 0)),
            pl.BlockSpec((1, 1), lambda i, c: (0, 0)),
        ],
        out_shape=[
            jax.ShapeDtypeStruct((N_TOK, 1), jnp.int32),
            jax.ShapeDtypeStruct((1, 1), jnp.float32),
        ],
        scratch_shapes=[
            pltpu.VMEM((BT, 1), jnp.float32),
            pltpu.VMEM((BT, 1), jnp.int32),
        ],
        compiler_params=pltpu.CompilerParams(
            dimension_semantics=("arbitrary", "arbitrary")),
    )(xf, a, b2, emb)


_NW = 32
_BPW = N_TOK // _NW


def _make_gather():
    mesh = plsc.VectorSubcoreMesh(core_axis_name="c", subcore_axis_name="s")

    @functools.partial(
        pl.kernel, mesh=mesh,
        compiler_params=pltpu.CompilerParams(use_tc_tiling_on_sc=False),
        out_type=jax.ShapeDtypeStruct((N_TOK, DIM), jnp.float32),
        scratch_types=[
            pltpu.VMEM((_BPW,), jnp.int32),
            pltpu.VMEM((_BPW, DIM), jnp.float32),
            pltpu.VMEM((_BPW, DIM), jnp.float32),
            pltpu.SemaphoreType.DMA,
        ],
    )
    def gather_st(table_hbm, idx_hbm, x_hbm, out_hbm, idx_v, rows_v, x_v, sem):
        wid = lax.axis_index("s") * 2 + lax.axis_index("c")
        base = wid * _BPW
        pltpu.sync_copy(idx_hbm.at[pl.ds(base, _BPW)], idx_v)
        pltpu.async_copy(table_hbm.at[idx_v], rows_v, sem).wait()
        pltpu.sync_copy(x_hbm.at[pl.ds(base, _BPW)], x_v)

        def body(i, carry):
            for c in range(DIM // 16):
                xv = x_v[i, pl.ds(c * 16, 16)]
                qv = rows_v[i, pl.ds(c * 16, 16)]
                rows_v[i, pl.ds(c * 16, 16)] = xv + (qv - xv)
            return carry

        lax.fori_loop(0, _BPW, body, 0)
        pltpu.sync_copy(rows_v, out_hbm.at[pl.ds(base, _BPW)])

    return gather_st


_gather_st = _make_gather()


def kernel(inputs, embedding):
    x5 = jnp.transpose(inputs, (0, 2, 3, 4, 1))
    xf = x5.reshape(-1, DIM)
    a = jnp.sum(xf ** 2, axis=1, keepdims=True)
    b = jnp.sum(embedding ** 2, axis=1)
    idx2, loss_sum = _argmin_call(xf, a, b.reshape(1, N_CODE), embedding)
    idx = idx2.reshape(N_TOK)
    qst = _gather_st(embedding, idx, xf)
    m = loss_sum[0, 0] / (N_TOK * DIM)
    loss = m + 0.25 * m
    out = qst.reshape(x5.shape).transpose(0, 4, 1, 2, 3)
    return out, loss, idx.reshape(x5.shape[:-1])


# speedup vs baseline: 1.7742x; 1.7742x over previous
import functools

import jax
import jax.numpy as jnp
from jax import lax
from jax.experimental import pallas as pl
from jax.experimental.pallas import tpu as pltpu
from jax.experimental.pallas import tpu_sc as plsc

N_TOK = 8192
N_CODE = 8192
DIM = 32

BT = 512
WC = 2048
GT = N_TOK // BT
GC = N_CODE // WC


def _argmin_body(x_ref, a_ref, b_ref, e_ref, idx_ref, loss_ref, mn_ref, mi_ref):
    c = pl.program_id(1)

    @pl.when(c == 0)
    def _init():
        mn_ref[...] = jnp.full((BT, 1), jnp.inf, jnp.float32)
        mi_ref[...] = jnp.zeros((BT, 1), jnp.int32)

    x = x_ref[...]
    a = a_ref[...]
    b = b_ref[...]
    e = e_ref[...]
    cmat = lax.dot_general(x, e, (((1,), (1,)), ((), ())),
                           preferred_element_type=jnp.float32)
    d = (a + b) - 2.0 * cmat
    m = jnp.min(d, axis=1, keepdims=True)
    cols = lax.broadcasted_iota(jnp.int32, (BT, WC), 1) + c * WC
    im = jnp.min(jnp.where(d == m, cols, jnp.int32(2**30)),
                 axis=1, keepdims=True)
    mn = mn_ref[...]
    mi = mi_ref[...]
    upd = m < mn
    mn2 = jnp.where(upd, m, mn)
    mi2 = jnp.where(upd, im, mi)
    mn_ref[...] = mn2
    mi_ref[...] = mi2
    idx_ref[...] = mi2

    @pl.when(jnp.logical_and(pl.program_id(0) == 0, c == 0))
    def _zero():
        loss_ref[...] = jnp.zeros((1, 1), jnp.float32)

    @pl.when(c == GC - 1)
    def _acc():
        loss_ref[...] = loss_ref[...] + jnp.sum(mn2, keepdims=True)


@jax.jit
def _argmin_call(xf, a, b2, emb):
    return pl.pallas_call(
        _argmin_body,
        grid=(GT, GC),
        in_specs=[
            pl.BlockSpec((BT, DIM), lambda i, c: (i, 0)),
            pl.BlockSpec((BT, 1), lambda i, c: (i, 0)),
            pl.BlockSpec((1, WC), lambda i, c: (0, c)),
            pl.BlockSpec((WC, DIM), lambda i, c: (c, 0)),
        ],
        out_specs=[
            pl.BlockSpec((BT, 1), lambda i, c: (i, 0)),
            pl.BlockSpec((1, 1), lambda i, c: (0, 0)),
        ],
        out_shape=[
            jax.ShapeDtypeStruct((N_TOK, 1), jnp.int32),
            jax.ShapeDtypeStruct((1, 1), jnp.float32),
        ],
        scratch_shapes=[
            pltpu.VMEM((BT, 1), jnp.float32),
            pltpu.VMEM((BT, 1), jnp.int32),
        ],
        compiler_params=pltpu.CompilerParams(
            dimension_semantics=("arbitrary", "arbitrary")),
    )(xf, a, b2, emb)


_NW = 32
_BPW = N_TOK // _NW


def _make_gather():
    mesh = plsc.VectorSubcoreMesh(core_axis_name="c", subcore_axis_name="s")

    @functools.partial(
        pl.kernel, mesh=mesh,
        compiler_params=pltpu.CompilerParams(use_tc_tiling_on_sc=False),
        out_type=jax.ShapeDtypeStruct((N_TOK, DIM), jnp.float32),
        scratch_types=[
            pltpu.VMEM((_BPW,), jnp.int32),
            pltpu.VMEM((_BPW, DIM), jnp.float32),
            pltpu.VMEM((_BPW, DIM), jnp.float32),
            pltpu.SemaphoreType.DMA,
        ],
    )
    def gather_st(table_hbm, idx_hbm, x_hbm, out_hbm, idx_v, rows_v, x_v, sem):
        wid = lax.axis_index("s") * 2 + lax.axis_index("c")
        base = wid * _BPW
        pltpu.sync_copy(idx_hbm.at[pl.ds(base, _BPW)], idx_v)
        pltpu.async_copy(table_hbm.at[idx_v], rows_v, sem).wait()
        pltpu.sync_copy(x_hbm.at[pl.ds(base, _BPW)], x_v)

        def body(i, carry):
            for c in range(DIM // 16):
                xv = x_v[i, pl.ds(c * 16, 16)]
                qv = rows_v[i, pl.ds(c * 16, 16)]
                rows_v[i, pl.ds(c * 16, 16)] = xv + (qv - xv)
            return carry

        lax.fori_loop(0, _BPW, body, 0)
        pltpu.sync_copy(rows_v, out_hbm.at[pl.ds(base, _BPW)])

    return gather_st


_gather_st = _make_gather()


def kernel(inputs, embedding):
    x5 = jnp.transpose(inputs, (0, 2, 3, 4, 1))
    xf = x5.reshape(-1, DIM)
    a = jnp.sum(xf ** 2, axis=1, keepdims=True)
    b = jnp.sum(embedding ** 2, axis=1)
    idx2, loss_sum = _argmin_call(xf, a, b.reshape(1, N_CODE), embedding)
    idx = idx2.reshape(N_TOK)
    qst = _gather_st(embedding, idx, xf)
    m = loss_sum[0, 0] / (N_TOK * DIM)
    loss = m + 0.25 * m
    out = qst.reshape(x5.shape).transpose(0, 4, 1, 2, 3)
    return out, loss, idx.reshape(x5.shape[:-1])


# speedup vs baseline: 1.8915x; 1.0661x over previous
import functools

import jax
import jax.numpy as jnp
from jax import lax
from jax.experimental import pallas as pl
from jax.experimental.pallas import tpu as pltpu
from jax.experimental.pallas import tpu_sc as plsc

N_TOK = 8192
N_CODE = 8192
DIM = 32

BT = 1024
WC = 2048
GT = N_TOK // BT
GC = N_CODE // WC


def _argmin_body(x_ref, a_ref, b_ref, e_ref, idx_ref, loss_ref, mn_ref, mi_ref):
    c = pl.program_id(1)

    @pl.when(c == 0)
    def _init():
        mn_ref[...] = jnp.full((BT, 1), jnp.inf, jnp.float32)
        mi_ref[...] = jnp.zeros((BT, 1), jnp.int32)

    x = x_ref[...]
    a = a_ref[...]
    b = b_ref[...]
    e = e_ref[...]
    cmat = lax.dot_general(x, e, (((1,), (1,)), ((), ())),
                           preferred_element_type=jnp.float32)
    d = (a + b) - 2.0 * cmat
    m = jnp.min(d, axis=1, keepdims=True)
    cols = lax.broadcasted_iota(jnp.int32, (BT, WC), 1) + c * WC
    im = jnp.min(jnp.where(d == m, cols, jnp.int32(2**30)),
                 axis=1, keepdims=True)
    mn = mn_ref[...]
    mi = mi_ref[...]
    upd = m < mn
    mn2 = jnp.where(upd, m, mn)
    mi2 = jnp.where(upd, im, mi)
    mn_ref[...] = mn2
    mi_ref[...] = mi2
    idx_ref[...] = mi2

    @pl.when(jnp.logical_and(pl.program_id(0) == 0, c == 0))
    def _zero():
        loss_ref[...] = jnp.zeros((1, 1), jnp.float32)

    @pl.when(c == GC - 1)
    def _acc():
        loss_ref[...] = loss_ref[...] + jnp.sum(mn2, keepdims=True)


@jax.jit
def _argmin_call(xf, a, b2, emb):
    return pl.pallas_call(
        _argmin_body,
        grid=(GT, GC),
        in_specs=[
            pl.BlockSpec((BT, DIM), lambda i, c: (i, 0)),
            pl.BlockSpec((BT, 1), lambda i, c: (i, 0)),
            pl.BlockSpec((1, WC), lambda i, c: (0, c)),
            pl.BlockSpec((WC, DIM), lambda i, c: (c, 0)),
        ],
        out_specs=[
            pl.BlockSpec((BT, 1), lambda i, c: (i, 0)),
            pl.BlockSpec((1, 1), lambda i, c: (0, 0)),
        ],
        out_shape=[
            jax.ShapeDtypeStruct((N_TOK, 1), jnp.int32),
            jax.ShapeDtypeStruct((1, 1), jnp.float32),
        ],
        scratch_shapes=[
            pltpu.VMEM((BT, 1), jnp.float32),
            pltpu.VMEM((BT, 1), jnp.int32),
        ],
        compiler_params=pltpu.CompilerParams(
            dimension_semantics=("arbitrary", "arbitrary")),
    )(xf, a, b2, emb)


_NW = 32
_BPW = N_TOK // _NW


def _make_gather():
    mesh = plsc.VectorSubcoreMesh(core_axis_name="c", subcore_axis_name="s")

    @functools.partial(
        pl.kernel, mesh=mesh,
        compiler_params=pltpu.CompilerParams(use_tc_tiling_on_sc=False),
        out_type=jax.ShapeDtypeStruct((N_TOK, DIM), jnp.float32),
        scratch_types=[
            pltpu.VMEM((_BPW,), jnp.int32),
            pltpu.VMEM((_BPW, DIM), jnp.float32),
            pltpu.VMEM((_BPW, DIM), jnp.float32),
            pltpu.SemaphoreType.DMA,
        ],
    )
    def gather_st(table_hbm, idx_hbm, x_hbm, out_hbm, idx_v, rows_v, x_v, sem):
        wid = lax.axis_index("s") * 2 + lax.axis_index("c")
        base = wid * _BPW
        pltpu.sync_copy(idx_hbm.at[pl.ds(base, _BPW)], idx_v)
        pltpu.async_copy(table_hbm.at[idx_v], rows_v, sem).wait()
        pltpu.sync_copy(x_hbm.at[pl.ds(base, _BPW)], x_v)

        def body(i, carry):
            for c in range(DIM // 16):
                xv = x_v[i, pl.ds(c * 16, 16)]
                qv = rows_v[i, pl.ds(c * 16, 16)]
                rows_v[i, pl.ds(c * 16, 16)] = xv + (qv - xv)
            return carry

        lax.fori_loop(0, _BPW, body, 0)
        pltpu.sync_copy(rows_v, out_hbm.at[pl.ds(base, _BPW)])

    return gather_st


_gather_st = _make_gather()


def kernel(inputs, embedding):
    x5 = jnp.transpose(inputs, (0, 2, 3, 4, 1))
    xf = x5.reshape(-1, DIM)
    a = jnp.sum(xf ** 2, axis=1, keepdims=True)
    b = jnp.sum(embedding ** 2, axis=1)
    idx2, loss_sum = _argmin_call(xf, a, b.reshape(1, N_CODE), embedding)
    idx = idx2.reshape(N_TOK)
    qst = _gather_st(embedding, idx, xf)
    m = loss_sum[0, 0] / (N_TOK * DIM)
    loss = m + 0.25 * m
    out = qst.reshape(x5.shape).transpose(0, 4, 1, 2, 3)
    return out, loss, idx.reshape(x5.shape[:-1])
